# baseline (speedup 1.0000x reference)
import functools

import jax
import jax.numpy as jnp
from jax import lax
from jax.experimental import pallas as pl
from jax.experimental.pallas import tpu as pltpu
from jax.experimental.pallas import tpu_sc as plsc

N = 10000
E = 160000
D_IN = 256
D_H = 512
CH = 128
BATCH = 128
NSUB = 16
NCORE = 2
EP = ((E + 2 * NCORE * NSUB * BATCH - 1) // (2 * NCORE * NSUB * BATCH)) * (2 * NCORE * NSUB * BATCH)
PAD = EP - E
ROWS_PER_TILE = EP // (NSUB * BATCH)
DEG_ROWS_PER_TILE = EP // (NCORE * NSUB * BATCH)
ACC_ROWS = ((N + 64 + NSUB * 8 - 1) // (NSUB * 8)) * (NSUB * 8)
ZROWS_PER_TILE = ACC_ROWS // NSUB
NBLK = 10
BN = N // NBLK


def _mesh():
    return plsc.VectorSubcoreMesh(core_axis_name="c", subcore_axis_name="s")


@functools.partial(
    pl.kernel,
    out_type=jax.ShapeDtypeStruct((NCORE, ACC_ROWS, 16), jnp.float32),
    mesh=_mesh(),
    scratch_types=[
        pltpu.VMEM((DEG_ROWS_PER_TILE, 128), jnp.int32),
        pltpu.VMEM((BATCH, 16), jnp.float32),
        pltpu.VMEM((BATCH, 16), jnp.float32),
        pltpu.VMEM_SHARED((ACC_ROWS, 16), jnp.float32),
        pltpu.SemaphoreType.DMA,
    ],
)
def _deg_kernel(dstrow, out, dst_v, ones_v, zb, acc, sem):
    core = lax.axis_index("c")
    sub = lax.axis_index("s")

    def fill(r, carry):
        ones_v[r, :] = jnp.full((16,), 1.0, jnp.float32)
        zb[r, :] = jnp.zeros((16,), jnp.float32)
        return carry

    lax.fori_loop(0, BATCH, fill, 0)

    r0 = sub * ZROWS_PER_TILE
    for k in range(5):
        cnt = BATCH if k < 4 else ZROWS_PER_TILE - 4 * BATCH
        pltpu.sync_copy(zb.at[pl.ds(0, cnt)], acc.at[pl.ds(r0 + k * BATCH, cnt)])
    plsc.subcore_barrier()

    row0 = core * (EP // (NCORE * BATCH)) + sub * DEG_ROWS_PER_TILE
    pltpu.sync_copy(dstrow.at[pl.ds(row0, DEG_ROWS_PER_TILE)], dst_v)

    def body(b, carry):
        pltpu.sync_copy(ones_v, acc.at[dst_v.at[b]], add=True)
        return carry

    lax.fori_loop(0, DEG_ROWS_PER_TILE, body, 0)
    plsc.subcore_barrier()

    pltpu.sync_copy(acc.at[pl.ds(r0, ZROWS_PER_TILE)],
                    out.at[core, pl.ds(r0, ZROWS_PER_TILE), :])


def _make_agg(base, nc, chain=False):
    HALF = ROWS_PER_TILE // 2

    def agg_chain(fp_hbm, srcrow, dstrow, prev_hbm, out_hbm, src_v, dst_v,
                  buf_a, buf_b, acc, sem_a, sem_b):
        return _agg_impl(fp_hbm, srcrow, dstrow, out_hbm, src_v, dst_v,
                         buf_a, buf_b, acc, sem_a, sem_b)

    def agg_plain(fp_hbm, srcrow, dstrow, out_hbm, src_v, dst_v,
                  buf_a, buf_b, acc, sem_a, sem_b):
        return _agg_impl(fp_hbm, srcrow, dstrow, out_hbm, src_v, dst_v,
                         buf_a, buf_b, acc, sem_a, sem_b)

    def _agg_impl(fp_hbm, srcrow, dstrow, out_hbm, src_v, dst_v, buf_a, buf_b,
                  acc, sem_a, sem_b):
        core = lax.axis_index("c")
        sub = lax.axis_index("s")

        def start_a(b):
            pltpu.async_copy(fp_hbm.at[src_v.at[b]], buf_a, sem_a)

        def start_b(b):
            pltpu.async_copy(fp_hbm.at[src_v.at[b]], buf_b, sem_b)

        def wait_a():
            pltpu.make_async_copy(fp_hbm.at[src_v.at[0]], buf_a, sem_a).wait()

        def wait_b():
            pltpu.make_async_copy(fp_hbm.at[src_v.at[0]], buf_b, sem_b).wait()

        off = (base + core) * N

        def zrow(r, carry):
            for l in range(CH // 16):
                buf_a[r, pl.ds(l * 16, 16)] = jnp.zeros((16,), jnp.float32)
            return carry

        lax.fori_loop(0, BATCH, zrow, 0)
        r0 = sub * ZROWS_PER_TILE
        for k in range(5):
            cnt = BATCH if k < 4 else ZROWS_PER_TILE - 4 * BATCH
            pltpu.sync_copy(buf_a.at[pl.ds(0, cnt)],
                            acc.at[pl.ds(r0 + k * BATCH, cnt)])
        plsc.subcore_barrier()

        for half in range(2):
            rowb = sub * ROWS_PER_TILE + half * HALF
            pltpu.sync_copy(srcrow.at[pl.ds(rowb, HALF)], src_v)
            pltpu.sync_copy(dstrow.at[pl.ds(rowb, HALF)], dst_v)

            def addoff(r, carry):
                for l in range(8):
                    sl = src_v[r, pl.ds(l * 16, 16)]
                    src_v[r, pl.ds(l * 16, 16)] = sl + off
                return carry

            lax.fori_loop(0, HALF, addoff, 0)

            start_a(0)

            def body(i, carry):
                b = 2 * i
                start_b(b + 1)
                wait_a()
                pltpu.sync_copy(buf_a, acc.at[dst_v.at[b]], add=True)
                start_a(lax.rem(b + 2, HALF))
                wait_b()
                pltpu.sync_copy(buf_b, acc.at[dst_v.at[b + 1]], add=True)
                return carry

            lax.fori_loop(0, HALF // 2, body, 0)
            wait_a()
        plsc.subcore_barrier()

        pltpu.sync_copy(acc.at[pl.ds(r0, ZROWS_PER_TILE)],
                        out_hbm.at[core, pl.ds(r0, ZROWS_PER_TILE), :])

    return pl.kernel(
        agg_chain if chain else agg_plain,
        out_type=jax.ShapeDtypeStruct((NCORE, ACC_ROWS, CH), jnp.float32),
        mesh=_mesh(),
        scratch_types=[
            pltpu.VMEM((HALF, 128), jnp.int32),
            pltpu.VMEM((HALF, 128), jnp.int32),
            pltpu.VMEM((BATCH, CH), jnp.float32),
            pltpu.VMEM((BATCH, CH), jnp.float32),
            pltpu.VMEM_SHARED((ACC_ROWS, CH), jnp.float32),
            pltpu.SemaphoreType.DMA,
            pltpu.SemaphoreType.DMA,
        ],
    )


_aggp0_2 = _make_agg(0, 2)
_aggp0_4 = _make_agg(0, 4)
_aggp2_4 = _make_agg(2, 4, chain=True)


def _dinv_body(degp_ref, dinv_ref):
    s = degp_ref[0] + degp_ref[1] + 1.0
    dinv_ref[...] = lax.rsqrt(s)


_dinv_call = pl.pallas_call(
    _dinv_body,
    out_shape=jax.ShapeDtypeStruct((ACC_ROWS, 16), jnp.float32),
)


def _scale0_body(x_ref, dinv_ref, out_ref):
    out_ref[...] = x_ref[...] * dinv_ref[:, 0:1]


_scale0_call = pl.pallas_call(
    _scale0_body,
    grid=(NBLK, D_IN // CH),
    in_specs=[
        pl.BlockSpec((BN, CH), lambda i, c: (i, c)),
        pl.BlockSpec((BN, 16), lambda i, c: (i, 0)),
    ],
    out_specs=pl.BlockSpec((BN, CH), lambda i, c: (c * NBLK + i, 0)),
    out_shape=jax.ShapeDtypeStruct(((D_IN // CH) * N, CH), jnp.float32),
)


def _make_k1(nc):
    def body(raw_ref, fp_ref, dinv_ref, w_ref, b_ref, h_ref, st_ref):
        i = pl.program_id(0)
        c = pl.program_id(1)
        dv = dinv_ref[:, 0:1]
        aggf = (raw_ref[0] + fp_ref[...]) * dv
        part = jnp.dot(aggf, w_ref[...], preferred_element_type=jnp.float32)

        @pl.when(c == 0)
        def _():
            h_ref[...] = part

        @pl.when(c > 0)
        def _():
            h_ref[...] = h_ref[...] + part

        @pl.when(c == nc - 1)
        def _():
            h = h_ref[...] + b_ref[...]
            h_ref[...] = h

            @pl.when(i == 0)
            def _():
                st_ref[...] = jnp.zeros_like(st_ref)

            st_ref[0:1, :] = st_ref[0:1, :] + jnp.sum(h, axis=0, keepdims=True)
            st_ref[1:2, :] = st_ref[1:2, :] + jnp.sum(h * h, axis=0, keepdims=True)

    return pl.pallas_call(
        body,
        grid=(NBLK, nc),
        in_specs=[
            pl.BlockSpec((1, BN, CH), lambda i, c: (c, i, 0)),
            pl.BlockSpec((BN, CH), lambda i, c: (c * NBLK + i, 0)),
            pl.BlockSpec((BN, 16), lambda i, c: (i, 0)),
            pl.BlockSpec((CH, D_H), lambda i, c: (c, 0)),
            pl.BlockSpec((1, D_H), lambda i, c: (0, 0)),
        ],
        out_specs=[
            pl.BlockSpec((BN, D_H), lambda i, c: (i, 0)),
            pl.BlockSpec((8, D_H), lambda i, c: (0, 0)),
        ],
        out_shape=[
            jax.ShapeDtypeStruct((N, D_H), jnp.float32),
            jax.ShapeDtypeStruct((8, D_H), jnp.float32),
        ],
    )


_k1_2 = _make_k1(2)


def _k1a_body(raw_ref, fp_ref, dinv_ref, w_ref, hp_ref):
    c = pl.program_id(1)
    dv = dinv_ref[:, 0:1]
    aggf = (raw_ref[0] + fp_ref[...]) * dv
    part = jnp.dot(aggf, w_ref[...], preferred_element_type=jnp.float32)

    @pl.when(c == 0)
    def _():
        hp_ref[...] = part

    @pl.when(c > 0)
    def _():
        hp_ref[...] = hp_ref[...] + part


_k1a = pl.pallas_call(
    _k1a_body,
    grid=(NBLK, 2),
    in_specs=[
        pl.BlockSpec((1, BN, CH), lambda i, c: (c, i, 0)),
        pl.BlockSpec((BN, CH), lambda i, c: (c * NBLK + i, 0)),
        pl.BlockSpec((BN, 16), lambda i, c: (i, 0)),
        pl.BlockSpec((CH, D_H), lambda i, c: (c, 0)),
    ],
    out_specs=pl.BlockSpec((BN, D_H), lambda i, c: (i, 0)),
    out_shape=jax.ShapeDtypeStruct((N, D_H), jnp.float32),
)


def _k1b_body(raw_ref, fp_ref, dinv_ref, w_ref, b_ref, hp_ref, h_ref, st_ref):
    i = pl.program_id(0)
    c = pl.program_id(1)
    dv = dinv_ref[:, 0:1]
    aggf = (raw_ref[0] + fp_ref[...]) * dv
    part = jnp.dot(aggf, w_ref[...], preferred_element_type=jnp.float32)

    @pl.when(c == 0)
    def _():
        h_ref[...] = hp_ref[...] + part

    @pl.when(c == 1)
    def _():
        h = h_ref[...] + part + b_ref[...]
        h_ref[...] = h

        @pl.when(i == 0)
        def _():
            st_ref[...] = jnp.zeros_like(st_ref)

        st_ref[0:1, :] = st_ref[0:1, :] + jnp.sum(h, axis=0, keepdims=True)
        st_ref[1:2, :] = st_ref[1:2, :] + jnp.sum(h * h, axis=0, keepdims=True)


_k1b = pl.pallas_call(
    _k1b_body,
    grid=(NBLK, 2),
    in_specs=[
        pl.BlockSpec((1, BN, CH), lambda i, c: (c, i, 0)),
        pl.BlockSpec((BN, CH), lambda i, c: ((c + 2) * NBLK + i, 0)),
        pl.BlockSpec((BN, 16), lambda i, c: (i, 0)),
        pl.BlockSpec((CH, D_H), lambda i, c: (c + 2, 0)),
        pl.BlockSpec((1, D_H), lambda i, c: (0, 0)),
        pl.BlockSpec((BN, D_H), lambda i, c: (i, 0)),
    ],
    out_specs=[
        pl.BlockSpec((BN, D_H), lambda i, c: (i, 0)),
        pl.BlockSpec((8, D_H), lambda i, c: (0, 0)),
    ],
    out_shape=[
        jax.ShapeDtypeStruct((N, D_H), jnp.float32),
        jax.ShapeDtypeStruct((8, D_H), jnp.float32),
    ],
)


def _make_k2(emit_fp, col, alias):
    def body(h_ref, st_ref, g_ref, be_ref, dinv_ref, *rest):
        rest = list(rest)
        if alias:
            rest.pop(0)
        y_ref = rest[0]
        mean = st_ref[0:1, :] * (1.0 / N)
        var = st_ref[1:2, :] * (1.0 / N) - mean * mean
        inv = lax.rsqrt(var + 1e-5)
        y = (h_ref[...] - mean) * inv * g_ref[...] + be_ref[...]
        y = jnp.maximum(y, 0.0)
        y_ref[...] = y
        if emit_fp:
            fp_ref = rest[1]
            dv = dinv_ref[:, 0:1]
            for c in range(D_H // CH):
                fp_ref[c] = y[:, c * CH:(c + 1) * CH] * dv

    in_specs = [
        pl.BlockSpec((BN, D_H), lambda i: (i, 0)),
        pl.BlockSpec((8, D_H), lambda i: (0, 0)),
        pl.BlockSpec((1, D_H), lambda i: (0, 0)),
        pl.BlockSpec((1, D_H), lambda i: (0, 0)),
        pl.BlockSpec((BN, 16), lambda i: (i, 0)),
    ]
    if alias:
        in_specs.append(pl.BlockSpec((8, D_H), lambda i: (0, 0)))

    out_specs = [pl.BlockSpec((BN, D_H), lambda i: (i, col))]
    out_shape = [jax.ShapeDtypeStruct((N, 3 * D_H), jnp.float32)]
    if emit_fp:
        out_specs.append(pl.BlockSpec((D_H // CH, BN, CH), lambda i: (0, i, 0)))
        out_shape.append(jax.ShapeDtypeStruct((D_H // CH, N, CH), jnp.float32))

    return pl.pallas_call(
        body,
        grid=(NBLK,),
        in_specs=in_specs,
        out_specs=out_specs,
        out_shape=out_shape,
        input_output_aliases={5: 0} if alias else {},
    )


_k2_first = _make_k2(True, 0, False)
_k2_mid1 = _make_k2(True, 1, True)
_k2_last = _make_k2(False, 2, True)


def kernel(x, edge_index, W0, b0, g0, be0, W1, b1, g1, be1, W2, b2, g2, be2):
    src = edge_index[0].astype(jnp.int32)
    dst = edge_index[1].astype(jnp.int32)
    k = jnp.arange(PAD, dtype=jnp.int32)
    src_p = jnp.concatenate([src, k % N])
    dst_p = jnp.concatenate([dst, N + (k % 64)])
    srcrow = src_p.reshape(EP // 128, 128)
    dstrow = dst_p.reshape(EP // 128, 128)

    degp = _deg_kernel(dstrow)
    dinv = _dinv_call(degp)

    fp0 = _scale0_call(x, dinv)
    raw0 = _aggp0_2(fp0, srcrow, dstrow)
    h0, st0 = _k1_2(raw0, fp0, dinv, W0, b0.reshape(1, D_H))
    slab, fpn0 = _k2_first(h0, st0, g0.reshape(1, D_H), be0.reshape(1, D_H),
                           dinv)

    fpn0f = fpn0.reshape((D_H // CH) * N, CH)
    rawa1 = _aggp0_4(fpn0f, srcrow, dstrow)
    rawb1 = _aggp2_4(fpn0f, srcrow, dstrow, rawa1)
    hp1 = _k1a(rawa1, fpn0f, dinv, W1)
    h1, st1 = _k1b(rawb1, fpn0f, dinv, W1, b1.reshape(1, D_H), hp1)
    slab, fpn1 = _k2_mid1(h1, st1, g1.reshape(1, D_H), be1.reshape(1, D_H),
                          dinv, slab)

    fpn1f = fpn1.reshape((D_H // CH) * N, CH)
    rawa2 = _aggp0_4(fpn1f, srcrow, dstrow)
    rawb2 = _aggp2_4(fpn1f, srcrow, dstrow, rawa2)
    hp2 = _k1a(rawa2, fpn1f, dinv, W2)
    h2, st2 = _k1b(rawb2, fpn1f, dinv, W2, b2.reshape(1, D_H), hp2)
    (slab,) = _k2_last(h2, st2, g2.reshape(1, D_H), be2.reshape(1, D_H),
                       dinv, slab)

    return slab


# speedup vs baseline: 1.0155x; 1.0155x over previous
import functools

import jax
import jax.numpy as jnp
from jax import lax
from jax.experimental import pallas as pl
from jax.experimental.pallas import tpu as pltpu
from jax.experimental.pallas import tpu_sc as plsc

N = 10000
E = 160000
D_IN = 256
D_H = 512
CH = 128
BATCH = 128
NSUB = 16
NCORE = 2
EP = ((E + 2 * NCORE * NSUB * BATCH - 1) // (2 * NCORE * NSUB * BATCH)) * (2 * NCORE * NSUB * BATCH)
PAD = EP - E
ROWS_PER_TILE = EP // (NSUB * BATCH)
DEG_ROWS_PER_TILE = EP // (NCORE * NSUB * BATCH)
ACC_ROWS = ((N + 64 + NSUB * 8 - 1) // (NSUB * 8)) * (NSUB * 8)
ZROWS_PER_TILE = ACC_ROWS // NSUB
NBLK = 10
BN = N // NBLK


def _mesh():
    return plsc.VectorSubcoreMesh(core_axis_name="c", subcore_axis_name="s")


@functools.partial(
    pl.kernel,
    out_type=jax.ShapeDtypeStruct((NCORE, ACC_ROWS, 16), jnp.float32),
    mesh=_mesh(),
    scratch_types=[
        pltpu.VMEM((DEG_ROWS_PER_TILE, 128), jnp.int32),
        pltpu.VMEM((BATCH, 16), jnp.float32),
        pltpu.VMEM((BATCH, 16), jnp.float32),
        pltpu.VMEM_SHARED((ACC_ROWS, 16), jnp.float32),
        pltpu.SemaphoreType.DMA,
    ],
)
def _deg_kernel(dstrow, out, dst_v, ones_v, zb, acc, sem):
    core = lax.axis_index("c")
    sub = lax.axis_index("s")

    def fill(r, carry):
        ones_v[r, :] = jnp.full((16,), 1.0, jnp.float32)
        zb[r, :] = jnp.zeros((16,), jnp.float32)
        return carry

    lax.fori_loop(0, BATCH, fill, 0)

    r0 = sub * ZROWS_PER_TILE
    for k in range(5):
        cnt = BATCH if k < 4 else ZROWS_PER_TILE - 4 * BATCH
        pltpu.sync_copy(zb.at[pl.ds(0, cnt)], acc.at[pl.ds(r0 + k * BATCH, cnt)])
    plsc.subcore_barrier()

    row0 = core * (EP // (NCORE * BATCH)) + sub * DEG_ROWS_PER_TILE
    pltpu.sync_copy(dstrow.at[pl.ds(row0, DEG_ROWS_PER_TILE)], dst_v)

    def body(b, carry):
        pltpu.sync_copy(ones_v, acc.at[dst_v.at[b]], add=True)
        return carry

    lax.fori_loop(0, DEG_ROWS_PER_TILE, body, 0)
    plsc.subcore_barrier()

    pltpu.sync_copy(acc.at[pl.ds(r0, ZROWS_PER_TILE)],
                    out.at[core, pl.ds(r0, ZROWS_PER_TILE), :])


def _make_agg(base, nc, chain=False):
    HALF = ROWS_PER_TILE // 2

    def agg_chain(fp_hbm, srcrow, dstrow, prev_hbm, out_hbm, src_v, dst_v,
                  buf_a, buf_b, acc, sem_a, sem_b):
        return _agg_impl(fp_hbm, srcrow, dstrow, out_hbm, src_v, dst_v,
                         buf_a, buf_b, acc, sem_a, sem_b)

    def agg_plain(fp_hbm, srcrow, dstrow, out_hbm, src_v, dst_v,
                  buf_a, buf_b, acc, sem_a, sem_b):
        return _agg_impl(fp_hbm, srcrow, dstrow, out_hbm, src_v, dst_v,
                         buf_a, buf_b, acc, sem_a, sem_b)

    def _agg_impl(fp_hbm, srcrow, dstrow, out_hbm, src_v, dst_v, buf_a, buf_b,
                  acc, sem_a, sem_b):
        core = lax.axis_index("c")
        sub = lax.axis_index("s")

        def start_a(b):
            pltpu.async_copy(fp_hbm.at[src_v.at[b]], buf_a, sem_a)

        def start_b(b):
            pltpu.async_copy(fp_hbm.at[src_v.at[b]], buf_b, sem_b)

        def wait_a():
            pltpu.make_async_copy(fp_hbm.at[src_v.at[0]], buf_a, sem_a).wait()

        def wait_b():
            pltpu.make_async_copy(fp_hbm.at[src_v.at[0]], buf_b, sem_b).wait()

        off = (base + core) * N

        def zrow(r, carry):
            for l in range(CH // 16):
                buf_a[r, pl.ds(l * 16, 16)] = jnp.zeros((16,), jnp.float32)
            return carry

        lax.fori_loop(0, BATCH, zrow, 0)
        r0 = sub * ZROWS_PER_TILE
        for k in range(5):
            cnt = BATCH if k < 4 else ZROWS_PER_TILE - 4 * BATCH
            pltpu.sync_copy(buf_a.at[pl.ds(0, cnt)],
                            acc.at[pl.ds(r0 + k * BATCH, cnt)])
        plsc.subcore_barrier()

        for half in range(2):
            rowb = sub * ROWS_PER_TILE + half * HALF
            pltpu.sync_copy(srcrow.at[pl.ds(rowb, HALF)], src_v)
            pltpu.sync_copy(dstrow.at[pl.ds(rowb, HALF)], dst_v)

            def addoff(r, carry):
                for l in range(8):
                    sl = src_v[r, pl.ds(l * 16, 16)]
                    src_v[r, pl.ds(l * 16, 16)] = sl + off
                return carry

            lax.fori_loop(0, HALF, addoff, 0)

            start_a(0)

            def body(i, carry):
                b = 2 * i
                start_b(b + 1)
                wait_a()
                pltpu.sync_copy(buf_a, acc.at[dst_v.at[b]], add=True)
                start_a(lax.rem(b + 2, HALF))
                wait_b()
                pltpu.sync_copy(buf_b, acc.at[dst_v.at[b + 1]], add=True)
                return carry

            lax.fori_loop(0, HALF // 2, body, 0)
            wait_a()
        plsc.subcore_barrier()

        pltpu.sync_copy(acc.at[pl.ds(r0, ZROWS_PER_TILE)],
                        out_hbm.at[core, pl.ds(r0, ZROWS_PER_TILE), :])

    return pl.kernel(
        agg_chain if chain else agg_plain,
        out_type=jax.ShapeDtypeStruct((NCORE, ACC_ROWS, CH), jnp.float32),
        mesh=_mesh(),
        scratch_types=[
            pltpu.VMEM((HALF, 128), jnp.int32),
            pltpu.VMEM((HALF, 128), jnp.int32),
            pltpu.VMEM((BATCH, CH), jnp.float32),
            pltpu.VMEM((BATCH, CH), jnp.float32),
            pltpu.VMEM_SHARED((ACC_ROWS, CH), jnp.float32),
            pltpu.SemaphoreType.DMA,
            pltpu.SemaphoreType.DMA,
        ],
    )


_aggA = _make_agg(0, 2)
_aggB = _make_agg(0, 2, chain=True)


def _scale0_body(x_ref, degp_ref, fp_ref, dinv_ref):
    dv = lax.rsqrt(degp_ref[0] + degp_ref[1] + 1.0)
    dinv_ref[...] = dv
    fp_ref[...] = x_ref[...] * dv[:, 0:1]


_scale0_call = pl.pallas_call(
    _scale0_body,
    grid=(NBLK, D_IN // CH),
    in_specs=[
        pl.BlockSpec((BN, CH), lambda i, c: (i, c)),
        pl.BlockSpec((2, BN, 16), lambda i, c: (0, i, 0)),
    ],
    out_specs=[
        pl.BlockSpec((BN, CH), lambda i, c: (c * NBLK + i, 0)),
        pl.BlockSpec((BN, 16), lambda i, c: (i, 0)),
    ],
    out_shape=[
        jax.ShapeDtypeStruct(((D_IN // CH) * N, CH), jnp.float32),
        jax.ShapeDtypeStruct((N, 16), jnp.float32),
    ],
)


def _make_k1(nc):
    def body(raw_ref, fp_ref, dinv_ref, w_ref, b_ref, h_ref, st_ref):
        i = pl.program_id(0)
        c = pl.program_id(1)
        dv = dinv_ref[:, 0:1]
        aggf = (raw_ref[0] + fp_ref[...]) * dv
        part = jnp.dot(aggf, w_ref[...], preferred_element_type=jnp.float32)

        @pl.when(c == 0)
        def _():
            h_ref[...] = part

        @pl.when(c > 0)
        def _():
            h_ref[...] = h_ref[...] + part

        @pl.when(c == nc - 1)
        def _():
            h = h_ref[...] + b_ref[...]
            h_ref[...] = h

            @pl.when(i == 0)
            def _():
                st_ref[...] = jnp.zeros_like(st_ref)

            st_ref[0:1, :] = st_ref[0:1, :] + jnp.sum(h, axis=0, keepdims=True)
            st_ref[1:2, :] = st_ref[1:2, :] + jnp.sum(h * h, axis=0, keepdims=True)

    return pl.pallas_call(
        body,
        grid=(NBLK, nc),
        in_specs=[
            pl.BlockSpec((1, BN, CH), lambda i, c: (c, i, 0)),
            pl.BlockSpec((BN, CH), lambda i, c: (c * NBLK + i, 0)),
            pl.BlockSpec((BN, 16), lambda i, c: (i, 0)),
            pl.BlockSpec((CH, D_H), lambda i, c: (c, 0)),
            pl.BlockSpec((1, D_H), lambda i, c: (0, 0)),
        ],
        out_specs=[
            pl.BlockSpec((BN, D_H), lambda i, c: (i, 0)),
            pl.BlockSpec((8, D_H), lambda i, c: (0, 0)),
        ],
        out_shape=[
            jax.ShapeDtypeStruct((N, D_H), jnp.float32),
            jax.ShapeDtypeStruct((8, D_H), jnp.float32),
        ],
    )


_k1_2 = _make_k1(2)


def _k1a_body(raw_ref, fp_ref, dinv_ref, w_ref, hp_ref):
    c = pl.program_id(1)
    dv = dinv_ref[:, 0:1]
    aggf = (raw_ref[0] + fp_ref[...]) * dv
    part = jnp.dot(aggf, w_ref[...], preferred_element_type=jnp.float32)

    @pl.when(c == 0)
    def _():
        hp_ref[...] = part

    @pl.when(c > 0)
    def _():
        hp_ref[...] = hp_ref[...] + part


_k1a = pl.pallas_call(
    _k1a_body,
    grid=(NBLK, 2),
    in_specs=[
        pl.BlockSpec((1, BN, CH), lambda i, c: (c, i, 0)),
        pl.BlockSpec((BN, CH), lambda i, c: (c * NBLK + i, 0)),
        pl.BlockSpec((BN, 16), lambda i, c: (i, 0)),
        pl.BlockSpec((CH, D_H), lambda i, c: (c, 0)),
    ],
    out_specs=pl.BlockSpec((BN, D_H), lambda i, c: (i, 0)),
    out_shape=jax.ShapeDtypeStruct((N, D_H), jnp.float32),
)


def _k1b_body(raw_ref, fp_ref, dinv_ref, w_ref, b_ref, hp_ref, h_ref, st_ref):
    i = pl.program_id(0)
    c = pl.program_id(1)
    dv = dinv_ref[:, 0:1]
    aggf = (raw_ref[0] + fp_ref[...]) * dv
    part = jnp.dot(aggf, w_ref[...], preferred_element_type=jnp.float32)

    @pl.when(c == 0)
    def _():
        h_ref[...] = hp_ref[...] + part

    @pl.when(c == 1)
    def _():
        h = h_ref[...] + part + b_ref[...]
        h_ref[...] = h

        @pl.when(i == 0)
        def _():
            st_ref[...] = jnp.zeros_like(st_ref)

        st_ref[0:1, :] = st_ref[0:1, :] + jnp.sum(h, axis=0, keepdims=True)
        st_ref[1:2, :] = st_ref[1:2, :] + jnp.sum(h * h, axis=0, keepdims=True)


_k1b = pl.pallas_call(
    _k1b_body,
    grid=(NBLK, 2),
    in_specs=[
        pl.BlockSpec((1, BN, CH), lambda i, c: (c, i, 0)),
        pl.BlockSpec((BN, CH), lambda i, c: (c * NBLK + i, 0)),
        pl.BlockSpec((BN, 16), lambda i, c: (i, 0)),
        pl.BlockSpec((CH, D_H), lambda i, c: (c + 2, 0)),
        pl.BlockSpec((1, D_H), lambda i, c: (0, 0)),
        pl.BlockSpec((BN, D_H), lambda i, c: (i, 0)),
    ],
    out_specs=[
        pl.BlockSpec((BN, D_H), lambda i, c: (i, 0)),
        pl.BlockSpec((8, D_H), lambda i, c: (0, 0)),
    ],
    out_shape=[
        jax.ShapeDtypeStruct((N, D_H), jnp.float32),
        jax.ShapeDtypeStruct((8, D_H), jnp.float32),
    ],
)


def _make_k2h(half, col, alias):
    HW = D_H // 2

    def body(h_ref, st_ref, g_ref, be_ref, dinv_ref, *rest):
        rest = list(rest)
        if alias:
            rest.pop(0)
        y_ref, fp_ref = rest
        mean = st_ref[0:1, :] * (1.0 / N)
        var = st_ref[1:2, :] * (1.0 / N) - mean * mean
        inv = lax.rsqrt(var + 1e-5)
        y = (h_ref[...] - mean) * inv * g_ref[...] + be_ref[...]
        y = jnp.maximum(y, 0.0)
        y_ref[...] = y
        dv = dinv_ref[:, 0:1]
        fp_ref[0] = y[:, 0:CH] * dv
        fp_ref[1] = y[:, CH:2 * CH] * dv

    in_specs = [
        pl.BlockSpec((BN, HW), lambda i: (i, half)),
        pl.BlockSpec((8, HW), lambda i: (0, half)),
        pl.BlockSpec((1, HW), lambda i: (0, half)),
        pl.BlockSpec((1, HW), lambda i: (0, half)),
        pl.BlockSpec((BN, 16), lambda i: (i, 0)),
    ]
    if alias:
        in_specs.append(pl.BlockSpec((8, 128), lambda i: (0, 0)))

    return pl.pallas_call(
        body,
        grid=(NBLK,),
        in_specs=in_specs,
        out_specs=[
            pl.BlockSpec((BN, HW), lambda i: (i, col)),
            pl.BlockSpec((2, BN, CH), lambda i: (0, i, 0)),
        ],
        out_shape=[
            jax.ShapeDtypeStruct((N, 3 * D_H), jnp.float32),
            jax.ShapeDtypeStruct((2, N, CH), jnp.float32),
        ],
        input_output_aliases={5: 0} if alias else {},
    )


_k2a0 = _make_k2h(0, 0, False)
_k2b0 = _make_k2h(1, 1, True)
_k2a1 = _make_k2h(0, 2, True)
_k2b1 = _make_k2h(1, 3, True)


def _k2last_body(h_ref, st_ref, g_ref, be_ref, _slab_ref, y_ref):
    mean = st_ref[0:1, :] * (1.0 / N)
    var = st_ref[1:2, :] * (1.0 / N) - mean * mean
    inv = lax.rsqrt(var + 1e-5)
    y = (h_ref[...] - mean) * inv * g_ref[...] + be_ref[...]
    y_ref[...] = jnp.maximum(y, 0.0)


_k2_last = pl.pallas_call(
    _k2last_body,
    grid=(NBLK,),
    in_specs=[
        pl.BlockSpec((BN, D_H), lambda i: (i, 0)),
        pl.BlockSpec((8, D_H), lambda i: (0, 0)),
        pl.BlockSpec((1, D_H), lambda i: (0, 0)),
        pl.BlockSpec((1, D_H), lambda i: (0, 0)),
        pl.BlockSpec((8, 128), lambda i: (0, 0)),
    ],
    out_specs=pl.BlockSpec((BN, D_H), lambda i: (i, 2)),
    out_shape=jax.ShapeDtypeStruct((N, 3 * D_H), jnp.float32),
    input_output_aliases={4: 0},
)


def kernel(x, edge_index, W0, b0, g0, be0, W1, b1, g1, be1, W2, b2, g2, be2):
    src = edge_index[0].astype(jnp.int32)
    dst = edge_index[1].astype(jnp.int32)
    k = jnp.arange(PAD, dtype=jnp.int32)
    src_p = jnp.concatenate([src, k % N])
    dst_p = jnp.concatenate([dst, N + (k % 64)])
    srcrow = src_p.reshape(EP // 128, 128)
    dstrow = dst_p.reshape(EP // 128, 128)

    degp = _deg_kernel(dstrow)
    fp0, dinv = _scale0_call(x, degp)
    raw0 = _aggA(fp0, srcrow, dstrow)
    h0, st0 = _k1_2(raw0, fp0, dinv, W0, b0.reshape(1, D_H))

    slab, fa1 = _k2a0(h0, st0, g0.reshape(1, D_H), be0.reshape(1, D_H), dinv)
    fa1f = fa1.reshape(2 * N, CH)
    ra1 = _aggA(fa1f, srcrow, dstrow)
    slab, fb1 = _k2b0(h0, st0, g0.reshape(1, D_H), be0.reshape(1, D_H),
                      dinv, slab)
    fb1f = fb1.reshape(2 * N, CH)
    rb1 = _aggB(fb1f, srcrow, dstrow, ra1)
    hp1 = _k1a(ra1, fa1f, dinv, W1)
    h1, st1 = _k1b(rb1, fb1f, dinv, W1, b1.reshape(1, D_H), hp1)

    slab, fa2 = _k2a1(h1, st1, g1.reshape(1, D_H), be1.reshape(1, D_H),
                      dinv, slab)
    fa2f = fa2.reshape(2 * N, CH)
    ra2 = _aggA(fa2f, srcrow, dstrow)
    slab, fb2 = _k2b1(h1, st1, g1.reshape(1, D_H), be1.reshape(1, D_H),
                      dinv, slab)
    fb2f = fb2.reshape(2 * N, CH)
    rb2 = _aggB(fb2f, srcrow, dstrow, ra2)
    hp2 = _k1a(ra2, fa2f, dinv, W2)
    h2, st2 = _k1b(rb2, fb2f, dinv, W2, b2.reshape(1, D_H), hp2)

    slab = _k2_last(h2, st2, g2.reshape(1, D_H), be2.reshape(1, D_H), slab)
    return slab
